# baseline (speedup 1.0000x reference)
import functools

import jax
import jax.numpy as jnp
from jax import lax
from jax.experimental import pallas as pl
from jax.experimental.pallas import tpu as pltpu
from jax.experimental.pallas import tpu_sc as plsc

B = 4096
T = 40960
V = 1000000
D = 32
L = 20
NC = 2
NS = 16
NW = NC * NS
BPW = B // NW
NE = BPW * L
NCHUNK = NE // 16
RPS = 8
EPS = RPS * L
NSUB = BPW // RPS
NT = L * D // 128


def _sc_body(idx0, off0, idx1, off1, tab0, tab1, out,
             offs_v, pos_v, msk_v, tok_v, big_v, asm_v, sem_i, sem_r, sem_o):
    wid = lax.axis_index("s") * NC + lax.axis_index("c")
    base = wid * BPW
    lane = lax.iota(jnp.int32, 16)

    def row_copy(tab_hbm, k, buf):
        e0 = k * EPS
        c1 = pltpu.make_async_copy(tab_hbm.at[tok_v.at[pl.ds(e0, 128)]],
                                   big_v.at[buf, pl.ds(0, 128)], sem_r)
        c2 = pltpu.make_async_copy(tab_hbm.at[tok_v.at[pl.ds(e0 + 128, 32)]],
                                   big_v.at[buf, pl.ds(128, 32)], sem_r)
        return c1, c2

    def out_copy(k, f, c, buf):
        return pltpu.make_async_copy(
            asm_v.at[buf, c],
            out.at[pl.ds(base + k * RPS, RPS),
                   pl.ds(f * L * D + c * 128, 128)], sem_o)

    for f, (idx_hbm, off_hbm, tab_hbm) in enumerate(
            ((idx0, off0, tab0), (idx1, off1, tab1))):
        pltpu.sync_copy(off_hbm, offs_v.at[pl.ds(0, B + 1)])

        def pos_body(i, carry):
            b, t = carry
            st = plsc.load_gather(offs_v, [base + b])
            en = plsc.load_gather(offs_v, [base + b + 1])
            pos = st + t
            valid = pos < en
            posc = jnp.minimum(pos, T - 1)
            pos_v[pl.ds(i * 16, 16)] = posc
            msk_v[pl.ds(i * 16, 16)] = jnp.where(valid, 1.0, 0.0)
            t2 = t + 16
            over = t2 >= L
            t_new = jnp.where(over, t2 - L, t2)
            b_new = b + jnp.where(over, 1, 0)
            return (b_new, t_new)

        lax.fori_loop(0, NCHUNK, pos_body,
                      (jnp.zeros((16,), jnp.int32), lane))

        def fire_idx(j, carry):
            pltpu.make_async_copy(idx_hbm.at[pos_v.at[pl.ds(j * 128, 128)]],
                                  tok_v.at[pl.ds(j * 128, 128)], sem_i).start()
            return carry

        lax.fori_loop(0, NE // 128, fire_idx, 0)

        def drain_idx(j, carry):
            pltpu.make_async_copy(idx_hbm.at[pos_v.at[pl.ds(j * 128, 128)]],
                                  tok_v.at[pl.ds(j * 128, 128)], sem_i).wait()
            return carry

        lax.fori_loop(0, NE // 128, drain_idx, 0)

        c1, c2 = row_copy(tab_hbm, 0, 0)
        c1.start()
        c2.start()

        def sub_body(k, carry):
            buf = k % 2
            c1, c2 = row_copy(tab_hbm, k, buf)
            c1.wait()
            c2.wait()

            @pl.when(k + 1 < NSUB)
            def _():
                n1, n2 = row_copy(tab_hbm, k + 1, 1 - buf)
                n1.start()
                n2.start()

            @pl.when(k >= 2)
            def _():
                for c in range(NT):
                    out_copy(k - 2, f, c, buf).wait()

            e0 = k * EPS

            def ext_body(i, carry2):
                s, t = carry2
                ii = i * 16 + lane
                m = msk_v[pl.ds(e0 + i * 16, 16)]
                cvec = lax.shift_right_logical(t, 2)
                lbase = lax.shift_left(t & 3, 5)
                for d in range(D):
                    val = plsc.load_gather(big_v, [jnp.full((16,), buf,
                                                            jnp.int32),
                                                   ii, jnp.full((16,), d,
                                                                jnp.int32)])
                    plsc.store_scatter(
                        asm_v,
                        [jnp.full((16,), buf, jnp.int32), cvec, s, lbase + d],
                        val * m)
                t2 = t + 16
                over = t2 >= L
                t_new = jnp.where(over, t2 - L, t2)
                s_new = s + jnp.where(over, 1, 0)
                return (s_new, t_new)

            lax.fori_loop(0, EPS // 16, ext_body,
                          (jnp.zeros((16,), jnp.int32), lane))

            for c in range(NT):
                out_copy(k, f, c, buf).start()
            return carry

        lax.fori_loop(0, NSUB, sub_body, 0)

        def drain_out(k, carry):
            for c in range(NT):
                out_copy(k, f, c, k % 2).wait()
            return carry

        lax.fori_loop(NSUB - 2, NSUB, drain_out, 0)


_sc_call = functools.partial(
    pl.kernel,
    mesh=plsc.VectorSubcoreMesh(core_axis_name="c", subcore_axis_name="s"),
    compiler_params=pltpu.CompilerParams(needs_layout_passes=False,
                                         use_tc_tiling_on_sc=True),
    out_type=jax.ShapeDtypeStruct((B, 2 * L * D), jnp.float32),
    scratch_types=[
        pltpu.VMEM((B + 128,), jnp.int32),
        pltpu.VMEM((NE,), jnp.int32),
        pltpu.VMEM((NE,), jnp.float32),
        pltpu.VMEM((NE,), jnp.int32),
        pltpu.VMEM((2, EPS, 128), jnp.float32),
        pltpu.VMEM((2, NT, RPS, 128), jnp.float32),
        pltpu.SemaphoreType.DMA,
        pltpu.SemaphoreType.DMA,
        pltpu.SemaphoreType.DMA,
    ],
)(_sc_body)


def kernel(indices_0, offsets_0, indices_1, offsets_1, table_0, table_1):
    return _sc_call(indices_0, offsets_0, indices_1, offsets_1,
                    jnp.tile(table_0, (1, 4)), jnp.tile(table_1, (1, 4)))


# speedup vs baseline: 1.3638x; 1.3638x over previous
import functools

import jax
import jax.numpy as jnp
from jax import lax
from jax.experimental import pallas as pl
from jax.experimental.pallas import tpu as pltpu
from jax.experimental.pallas import tpu_sc as plsc

B = 4096
T = 40960
V = 1000000
D = 32
L = 20
NC = 2
NS = 16
NW = NC * NS
BPW = B // NW
NE = BPW * L
NCHUNK = NE // 16
RPS = 8
EPS = RPS * L
NSUB = BPW // RPS
NT = L * D // 128


def _sc_body(idx0, off0, idx1, off1, tab0, tab1, out,
             offs_v, pos_v, msk_v, tok_v, col_v, big_v, asm_v,
             sem_i, sem_r, sem_o):
    wid = lax.axis_index("s") * NC + lax.axis_index("c")
    base = wid * BPW
    lane = lax.iota(jnp.int32, 16)

    def row_copy(tab_hbm, k, buf):
        e0 = k * EPS
        c1 = pltpu.make_async_copy(tab_hbm.at[tok_v.at[pl.ds(e0, 128)]],
                                   big_v.at[buf, pl.ds(0, 128)], sem_r)
        c2 = pltpu.make_async_copy(tab_hbm.at[tok_v.at[pl.ds(e0 + 128, 32)]],
                                   big_v.at[buf, pl.ds(128, 32)], sem_r)
        return c1, c2

    def out_copy(k, f, c, buf):
        return pltpu.make_async_copy(
            asm_v.at[buf, c],
            out.at[pl.ds(base + k * RPS, RPS),
                   pl.ds(f * L * D + c * 128, 128)], sem_o)

    for f, (idx_hbm, off_hbm, tab_hbm) in enumerate(
            ((idx0, off0, tab0), (idx1, off1, tab1))):
        pltpu.sync_copy(off_hbm, offs_v.at[pl.ds(0, B + 1)])

        def pos_body(i, carry):
            b, t = carry
            st = plsc.load_gather(offs_v, [base + b])
            en = plsc.load_gather(offs_v, [base + b + 1])
            pos = st + t
            valid = pos < en
            posc = jnp.minimum(pos, T - 1)
            pos_v[pl.ds(i * 16, 16)] = posc
            msk_v[pl.ds(i * 16, 16)] = jnp.where(valid, 1.0, 0.0)
            t2 = t + 16
            over = t2 >= L
            t_new = jnp.where(over, t2 - L, t2)
            b_new = b + jnp.where(over, 1, 0)
            return (b_new, t_new)

        lax.fori_loop(0, NCHUNK, pos_body,
                      (jnp.zeros((16,), jnp.int32), lane))

        def fire_idx(j, carry):
            pltpu.make_async_copy(idx_hbm.at[pos_v.at[pl.ds(j * 128, 128)]],
                                  tok_v.at[pl.ds(j * 128, 128)], sem_i).start()
            return carry

        lax.fori_loop(0, NE // 128, fire_idx, 0)

        def drain_idx(j, carry):
            pltpu.make_async_copy(idx_hbm.at[pos_v.at[pl.ds(j * 128, 128)]],
                                  tok_v.at[pl.ds(j * 128, 128)], sem_i).wait()
            return carry

        lax.fori_loop(0, NE // 128, drain_idx, 0)

        def rid_body(i, carry):
            v = tok_v[pl.ds(i * 16, 16)]
            tok_v[pl.ds(i * 16, 16)] = lax.shift_right_logical(v, 2)
            col_v[pl.ds(i * 16, 16)] = lax.shift_left(v & 3, 5)
            return carry

        lax.fori_loop(0, NCHUNK, rid_body, 0)

        c1, c2 = row_copy(tab_hbm, 0, 0)
        c1.start()
        c2.start()

        def sub_body(k, carry):
            buf = k % 2
            c1, c2 = row_copy(tab_hbm, k, buf)
            c1.wait()
            c2.wait()

            @pl.when(k + 1 < NSUB)
            def _():
                n1, n2 = row_copy(tab_hbm, k + 1, 1 - buf)
                n1.start()
                n2.start()

            @pl.when(k >= 2)
            def _():
                for c in range(NT):
                    out_copy(k - 2, f, c, buf).wait()

            e0 = k * EPS

            def ext_body(i, carry2):
                s, t = carry2
                ii = i * 16 + lane
                m = msk_v[pl.ds(e0 + i * 16, 16)]
                cb = col_v[pl.ds(e0 + i * 16, 16)]
                cvec = lax.shift_right_logical(t, 2)
                lbase = lax.shift_left(t & 3, 5)
                bufv = jnp.full((16,), buf, jnp.int32)
                for d in range(D):
                    val = plsc.load_gather(big_v, [bufv, ii, cb + d])
                    plsc.store_scatter(asm_v, [bufv, cvec, s, lbase + d],
                                       val * m)
                t2 = t + 16
                over = t2 >= L
                t_new = jnp.where(over, t2 - L, t2)
                s_new = s + jnp.where(over, 1, 0)
                return (s_new, t_new)

            lax.fori_loop(0, EPS // 16, ext_body,
                          (jnp.zeros((16,), jnp.int32), lane))

            for c in range(NT):
                out_copy(k, f, c, buf).start()
            return carry

        lax.fori_loop(0, NSUB, sub_body, 0)

        def drain_out(k, carry):
            for c in range(NT):
                out_copy(k, f, c, k % 2).wait()
            return carry

        lax.fori_loop(NSUB - 2, NSUB, drain_out, 0)


_sc_call = functools.partial(
    pl.kernel,
    mesh=plsc.VectorSubcoreMesh(core_axis_name="c", subcore_axis_name="s"),
    compiler_params=pltpu.CompilerParams(needs_layout_passes=False,
                                         use_tc_tiling_on_sc=True),
    out_type=jax.ShapeDtypeStruct((B, 2 * L * D), jnp.float32),
    scratch_types=[
        pltpu.VMEM((B + 128,), jnp.int32),
        pltpu.VMEM((NE,), jnp.int32),
        pltpu.VMEM((NE,), jnp.float32),
        pltpu.VMEM((NE,), jnp.int32),
        pltpu.VMEM((NE,), jnp.int32),
        pltpu.VMEM((2, EPS, 128), jnp.float32),
        pltpu.VMEM((2, NT, RPS, 128), jnp.float32),
        pltpu.SemaphoreType.DMA,
        pltpu.SemaphoreType.DMA,
        pltpu.SemaphoreType.DMA,
    ],
)(_sc_body)


def _pack(tab):
    flat = jax.lax.optimization_barrier(tab.reshape(-1))
    return flat.reshape(V // 4, 4 * D)


def kernel(indices_0, offsets_0, indices_1, offsets_1, table_0, table_1):
    return _sc_call(indices_0, offsets_0, indices_1, offsets_1,
                    _pack(table_0), _pack(table_1))


# speedup vs baseline: 1.4699x; 1.0778x over previous
import functools

import jax
import jax.numpy as jnp
from jax import lax
from jax.experimental import pallas as pl
from jax.experimental.pallas import tpu as pltpu
from jax.experimental.pallas import tpu_sc as plsc

B = 4096
T = 40960
V = 1000000
D = 32
L = 20
NC = 2
NS = 16
NW = NC * NS
BPW = B // NW
NE = BPW * L
NCHUNK = NE // 16
RPS = 8
EPS = RPS * L
NSUB = BPW // RPS
NT = L * D // 128


def _sc_body(idx_hbm, off_hbm, tab_hbm, out,
             offs_v, pos_v, msk_v, tok_v, col_v, big_v, asm_v,
             sem_i, sem_r, sem_o):
    wid = lax.axis_index("s") * NC + lax.axis_index("c")
    base = wid * BPW
    lane = lax.iota(jnp.int32, 16)

    def row_copy(tab_hbm, k, buf):
        e0 = k * EPS
        c1 = pltpu.make_async_copy(tab_hbm.at[tok_v.at[pl.ds(e0, 128)]],
                                   big_v.at[buf, pl.ds(0, 128)], sem_r)
        c2 = pltpu.make_async_copy(tab_hbm.at[tok_v.at[pl.ds(e0 + 128, 32)]],
                                   big_v.at[buf, pl.ds(128, 32)], sem_r)
        return c1, c2

    def out_copy(k, c, buf):
        return pltpu.make_async_copy(
            asm_v.at[buf, c],
            out.at[pl.ds(base + k * RPS, RPS), pl.ds(c * 128, 128)], sem_o)

    if True:
        pltpu.sync_copy(off_hbm, offs_v.at[pl.ds(0, B + 1)])

        def pos_body(i, carry):
            b, t = carry
            st = plsc.load_gather(offs_v, [base + b])
            en = plsc.load_gather(offs_v, [base + b + 1])
            pos = st + t
            valid = pos < en
            posc = jnp.minimum(pos, T - 1)
            pos_v[pl.ds(i * 16, 16)] = posc
            msk_v[pl.ds(i * 16, 16)] = jnp.where(valid, 1.0, 0.0)
            t2 = t + 16
            over = t2 >= L
            t_new = jnp.where(over, t2 - L, t2)
            b_new = b + jnp.where(over, 1, 0)
            return (b_new, t_new)

        lax.fori_loop(0, NCHUNK, pos_body,
                      (jnp.zeros((16,), jnp.int32), lane))

        def fire_idx(j, carry):
            pltpu.make_async_copy(idx_hbm.at[pos_v.at[pl.ds(j * 128, 128)]],
                                  tok_v.at[pl.ds(j * 128, 128)], sem_i).start()
            return carry

        lax.fori_loop(0, NE // 128, fire_idx, 0)

        def drain_idx(j, carry):
            pltpu.make_async_copy(idx_hbm.at[pos_v.at[pl.ds(j * 128, 128)]],
                                  tok_v.at[pl.ds(j * 128, 128)], sem_i).wait()
            return carry

        lax.fori_loop(0, NE // 128, drain_idx, 0)

        def rid_body(i, carry):
            v = tok_v[pl.ds(i * 16, 16)]
            tok_v[pl.ds(i * 16, 16)] = lax.shift_right_logical(v, 2)
            col_v[pl.ds(i * 16, 16)] = lax.shift_left(v & 3, 5)
            return carry

        lax.fori_loop(0, NCHUNK, rid_body, 0)

        c1, c2 = row_copy(tab_hbm, 0, 0)
        c1.start()
        c2.start()

        def sub_body(k, carry):
            buf = k % 2
            c1, c2 = row_copy(tab_hbm, k, buf)
            c1.wait()
            c2.wait()

            @pl.when(k + 1 < NSUB)
            def _():
                n1, n2 = row_copy(tab_hbm, k + 1, 1 - buf)
                n1.start()
                n2.start()

            @pl.when(k >= 2)
            def _():
                for c in range(NT):
                    out_copy(k - 2, c, buf).wait()

            e0 = k * EPS

            def ext_body(i, carry2):
                s, t = carry2
                ii = i * 16 + lane
                m = msk_v[pl.ds(e0 + i * 16, 16)]
                cb = col_v[pl.ds(e0 + i * 16, 16)]
                cvec = lax.shift_right_logical(t, 2)
                lbase = lax.shift_left(t & 3, 5)
                bufv = jnp.full((16,), buf, jnp.int32)
                for d in range(D):
                    val = plsc.load_gather(big_v, [bufv, ii, cb + d])
                    plsc.store_scatter(asm_v, [bufv, cvec, s, lbase + d],
                                       val * m)
                t2 = t + 16
                over = t2 >= L
                t_new = jnp.where(over, t2 - L, t2)
                s_new = s + jnp.where(over, 1, 0)
                return (s_new, t_new)

            lax.fori_loop(0, EPS // 16, ext_body,
                          (jnp.zeros((16,), jnp.int32), lane))

            for c in range(NT):
                out_copy(k, c, buf).start()
            return carry

        lax.fori_loop(0, NSUB, sub_body, 0)

        def drain_out(k, carry):
            for c in range(NT):
                out_copy(k, c, k % 2).wait()
            return carry

        lax.fori_loop(NSUB - 2, NSUB, drain_out, 0)


_sc_call = functools.partial(
    pl.kernel,
    mesh=plsc.VectorSubcoreMesh(core_axis_name="c", subcore_axis_name="s"),
    compiler_params=pltpu.CompilerParams(needs_layout_passes=False,
                                         use_tc_tiling_on_sc=True),
    out_type=jax.ShapeDtypeStruct((B, L * D), jnp.float32),
    scratch_types=[
        pltpu.VMEM((B + 128,), jnp.int32),
        pltpu.VMEM((NE,), jnp.int32),
        pltpu.VMEM((NE,), jnp.float32),
        pltpu.VMEM((NE,), jnp.int32),
        pltpu.VMEM((NE,), jnp.int32),
        pltpu.VMEM((2, EPS, 128), jnp.float32),
        pltpu.VMEM((2, NT, RPS, 128), jnp.float32),
        pltpu.SemaphoreType.DMA,
        pltpu.SemaphoreType.DMA,
        pltpu.SemaphoreType.DMA,
    ],
)(_sc_body)


def _pack(tab):
    flat = jax.lax.optimization_barrier(tab.reshape(-1))
    return flat.reshape(V // 4, 4 * D)


def kernel(indices_0, offsets_0, indices_1, offsets_1, table_0, table_1):
    d0 = _sc_call(indices_0, offsets_0, _pack(table_0))
    d1 = _sc_call(indices_1, offsets_1, _pack(table_1))
    return jnp.concatenate([d0, d1], axis=1)
